# Initial kernel scaffold; baseline (speedup 1.0000x reference)
import jax
import jax.numpy as jnp
from jax.experimental import pallas as pl


def kernel(x, edge_index, W1, a_s1, a_d1, b1, W2, a_s2, a_d2, b2, W3, a_s3, a_d3, b3):
    raise NotImplementedError("write your pallas kernel here")


# speedup vs baseline: 1.1040x; 1.1040x over previous
import functools

import jax
import jax.numpy as jnp
from jax.experimental import pallas as pl
from jax.experimental.pallas import tpu as pltpu

N = 10000
E = 320000


def _mm_body(x_ref, w_ref, as_ref, ad_ref, h_ref, e_ref):
    h = jnp.dot(x_ref[...], w_ref[...], preferred_element_type=jnp.float32)
    h_ref[...] = h
    e_s = jnp.sum(h * as_ref[...], axis=1)
    e_d = jnp.sum(h * ad_ref[...], axis=1)
    e_ref[0, 0, :] = e_s
    e_ref[0, 1, :] = e_d


@functools.partial(jax.jit, static_argnums=())
def _transform(x, W, a_s, a_d):
    n, din = x.shape
    dout = W.shape[1]
    blk = 2000
    grid = (n // blk,)
    h, e = pl.pallas_call(
        _mm_body,
        grid=grid,
        in_specs=[
            pl.BlockSpec((blk, din), lambda i: (i, 0)),
            pl.BlockSpec((din, dout), lambda i: (0, 0)),
            pl.BlockSpec((1, dout), lambda i: (0, 0)),
            pl.BlockSpec((1, dout), lambda i: (0, 0)),
        ],
        out_specs=[
            pl.BlockSpec((blk, dout), lambda i: (i, 0)),
            pl.BlockSpec((1, 2, blk), lambda i: (i, 0, 0)),
        ],
        out_shape=[
            jax.ShapeDtypeStruct((n, dout), jnp.float32),
            jax.ShapeDtypeStruct((n // blk, 2, blk), jnp.float32),
        ],
    )(x, W, a_s.reshape(1, -1), a_d.reshape(1, -1))
    e = jnp.transpose(e, (1, 0, 2)).reshape(2, n)
    return h, e[0], e[1]


def _gat_layer(x, src, dst, W, a_s, a_d, b):
    h, e_s, e_d = _transform(x, W, a_s, a_d)
    alpha = jax.nn.leaky_relu(e_s[src] + e_d[dst], negative_slope=0.2)
    amax = jax.ops.segment_max(alpha, dst, num_segments=N)
    amax = jnp.where(jnp.isfinite(amax), amax, 0.0)
    ex = jnp.exp(alpha - amax[dst])
    denom = jax.ops.segment_sum(ex, dst, num_segments=N)
    coef = ex / (denom[dst] + 1e-16)
    out = jax.ops.segment_sum(h[src] * coef[:, None], dst, num_segments=N)
    return out + b


def kernel(x, edge_index, W1, a_s1, a_d1, b1, W2, a_s2, a_d2, b2, W3, a_s3, a_d3, b3):
    src, dst = edge_index[0], edge_index[1]
    h = _gat_layer(x, src, dst, W1, a_s1, a_d1, b1)
    h = jax.nn.relu(h)
    h = _gat_layer(h, src, dst, W2, a_s2, a_d2, b2)
    h = jax.nn.relu(h)
    h = _gat_layer(h, src, dst, W3, a_s3, a_d3, b3)
    return jax.nn.log_softmax(h, axis=1)


# speedup vs baseline: 1.1058x; 1.0017x over previous
import jax
import jax.numpy as jnp
from jax.experimental import pallas as pl

NN = 10000
EE = 320000


def _mm_body(x_ref, w_ref, as_ref, ad_ref, h_ref, es_ref, ed_ref):
    h = jnp.dot(x_ref[...], w_ref[...], preferred_element_type=jnp.float32)
    h_ref[...] = h
    es_ref[0, 0, :] = jnp.sum(h * as_ref[...], axis=1)
    ed_ref[0, 0, :] = jnp.sum(h * ad_ref[...], axis=1)


def _transform1(x, W, a_s, a_d):
    n, din = x.shape
    dout = W.shape[1]
    blk = 2000
    nb = n // blk
    h, es, ed = pl.pallas_call(
        _mm_body,
        grid=(nb,),
        in_specs=[
            pl.BlockSpec((blk, din), lambda i: (i, 0)),
            pl.BlockSpec((din, dout), lambda i: (0, 0)),
            pl.BlockSpec((1, dout), lambda i: (0, 0)),
            pl.BlockSpec((1, dout), lambda i: (0, 0)),
        ],
        out_specs=[
            pl.BlockSpec((blk, dout), lambda i: (i, 0)),
            pl.BlockSpec((1, 1, blk), lambda i: (i, 0, 0)),
            pl.BlockSpec((1, 1, blk), lambda i: (i, 0, 0)),
        ],
        out_shape=[
            jax.ShapeDtypeStruct((n, dout), jnp.float32),
            jax.ShapeDtypeStruct((nb, 1, blk), jnp.float32),
            jax.ShapeDtypeStruct((nb, 1, blk), jnp.float32),
        ],
    )(x, W, a_s.reshape(1, -1), a_d.reshape(1, -1))
    return h, es.reshape(n), ed.reshape(n)


def _comb_mm_body(p_ref, b_ref, w_ref, as_ref, ad_ref, h_ref, es_ref, ed_ref):
    x = jax.nn.relu(p_ref[...] + b_ref[...])
    h = jnp.dot(x, w_ref[...], preferred_element_type=jnp.float32)
    h_ref[...] = h
    es_ref[0, 0, :] = jnp.sum(h * as_ref[...], axis=1)
    ed_ref[0, 0, :] = jnp.sum(h * ad_ref[...], axis=1)


def _transform_next(p, b, W, a_s, a_d):
    n, din = p.shape
    dout = W.shape[1]
    blk = 2000
    nb = n // blk
    h, es, ed = pl.pallas_call(
        _comb_mm_body,
        grid=(nb,),
        in_specs=[
            pl.BlockSpec((blk, din), lambda i: (i, 0)),
            pl.BlockSpec((1, din), lambda i: (0, 0)),
            pl.BlockSpec((din, dout), lambda i: (0, 0)),
            pl.BlockSpec((1, dout), lambda i: (0, 0)),
            pl.BlockSpec((1, dout), lambda i: (0, 0)),
        ],
        out_specs=[
            pl.BlockSpec((blk, dout), lambda i: (i, 0)),
            pl.BlockSpec((1, 1, blk), lambda i: (i, 0, 0)),
            pl.BlockSpec((1, 1, blk), lambda i: (i, 0, 0)),
        ],
        out_shape=[
            jax.ShapeDtypeStruct((n, dout), jnp.float32),
            jax.ShapeDtypeStruct((nb, 1, blk), jnp.float32),
            jax.ShapeDtypeStruct((nb, 1, blk), jnp.float32),
        ],
    )(p, b.reshape(1, -1), W, a_s.reshape(1, -1), a_d.reshape(1, -1))
    return h, es.reshape(n), ed.reshape(n)


def _final_body(p_ref, b_ref, o_ref):
    x = p_ref[...] + b_ref[...]
    m = jnp.max(x, axis=1, keepdims=True)
    o_ref[...] = (x - m) - jnp.log(
        jnp.sum(jnp.exp(x - m), axis=1, keepdims=True))


def _final(p, b):
    n, d = p.shape
    blk = 2000
    nb = n // blk
    return pl.pallas_call(
        _final_body,
        grid=(nb,),
        in_specs=[
            pl.BlockSpec((blk, d), lambda i: (i, 0)),
            pl.BlockSpec((1, d), lambda i: (0, 0)),
        ],
        out_specs=pl.BlockSpec((blk, d), lambda i: (i, 0)),
        out_shape=jax.ShapeDtypeStruct((n, d), jnp.float32),
    )(p, b.reshape(1, -1))


def _edge_phase(h, es, ed, src, dst):
    alpha = jax.nn.leaky_relu(es[src] + ed[dst], negative_slope=0.2)
    amax = jax.ops.segment_max(alpha, dst, num_segments=NN)
    amax = jnp.where(jnp.isfinite(amax), amax, 0.0)
    ex = jnp.exp(alpha - amax[dst])
    denom = jax.ops.segment_sum(ex, dst, num_segments=NN)
    coef = ex / (denom[dst] + 1e-16)
    return jax.ops.segment_sum(h[src] * coef[:, None], dst, num_segments=NN)


def kernel(x, edge_index, W1, a_s1, a_d1, b1, W2, a_s2, a_d2, b2, W3, a_s3, a_d3, b3):
    src, dst = edge_index[0], edge_index[1]

    h, es, ed = _transform1(x, W1, a_s1, a_d1)
    p = _edge_phase(h, es, ed, src, dst)

    h, es, ed = _transform_next(p, b1, W2, a_s2, a_d2)
    p = _edge_phase(h, es, ed, src, dst)

    h, es, ed = _transform_next(p, b2, W3, a_s3, a_d3)
    p = _edge_phase(h, es, ed, src, dst)

    return _final(p, b3)
